# baseline (speedup 1.0000x reference)
import functools

import jax
import jax.numpy as jnp
import numpy as np
from jax.experimental import pallas as pl
from jax.experimental.pallas import tpu as pltpu

_ROWS = 8

_ROT = (13, 15, 26, 6, 17, 29, 16, 24)
_TINY = np.float32(np.finfo(np.float32).tiny)


def _gumbel_argmax_block(logits_ref, out_ref, *, vocab, rows_per_block):
    b = pl.program_id(0)
    rows = rows_per_block
    row = jax.lax.broadcasted_iota(jnp.uint32, (rows, vocab), 0)
    col = jax.lax.broadcasted_iota(jnp.uint32, (rows, vocab), 1)
    i = (jnp.uint32(b) * jnp.uint32(rows) + row) * jnp.uint32(vocab) + col

    k1 = jnp.uint32(0)
    k2 = jnp.uint32(42)
    k3 = k1 ^ k2 ^ jnp.uint32(0x1BD11BDA)
    ks = (k1, k2, k3)
    x0 = jnp.zeros_like(i) + ks[0]
    x1 = i + ks[1]
    for g in range(5):
        rr = _ROT[:4] if g % 2 == 0 else _ROT[4:]
        for r in rr:
            x0 = x0 + x1
            x1 = (x1 << jnp.uint32(r)) | (x1 >> jnp.uint32(32 - r))
            x1 = x0 ^ x1
        x0 = x0 + ks[(g + 1) % 3]
        x1 = x1 + ks[(g + 2) % 3] + jnp.uint32(g + 1)
    bits = x0 ^ x1

    fb = (bits >> jnp.uint32(9)) | jnp.uint32(0x3F800000)
    u = jax.lax.bitcast_convert_type(fb, jnp.float32) - jnp.float32(1.0)
    u = jnp.maximum(_TINY, u)
    score = -jnp.log(-jnp.log(u)) + logits_ref[...]

    m = jnp.max(score, axis=1, keepdims=True)
    cand = jnp.where(score == m, col.astype(jnp.int32), jnp.int32(0x7FFFFFFF))
    out_ref[0, 0, :] = jnp.min(cand, axis=1)


def kernel(logits):
    batch, vocab = logits.shape
    assert batch % _ROWS == 0
    grid = batch // _ROWS
    out = pl.pallas_call(
        functools.partial(
            _gumbel_argmax_block, vocab=vocab, rows_per_block=_ROWS
        ),
        grid=(grid,),
        in_specs=[
            pl.BlockSpec((_ROWS, vocab), lambda b: (b, 0)),
        ],
        out_specs=pl.BlockSpec((1, 1, _ROWS), lambda b: (b, 0, 0)),
        out_shape=jax.ShapeDtypeStruct((grid, 1, _ROWS), jnp.int32),
        compiler_params=pltpu.CompilerParams(
            dimension_semantics=("parallel",),
        ),
    )(logits)
    return out.reshape(batch)


# speedup vs baseline: 1.0128x; 1.0128x over previous
import functools

import jax
import jax.numpy as jnp
import numpy as np
from jax.experimental import pallas as pl
from jax.experimental.pallas import tpu as pltpu

_ROWS = 8
_W = 512

_ROT = (13, 15, 26, 6, 17, 29, 16, 24)
_TINY = np.float32(np.finfo(np.float32).tiny)
_K1 = 0
_K2 = 42
_K3 = _K1 ^ _K2 ^ 0x1BD11BDA
_KS = (_K1, _K2, _K3)


def _score_chunk(i42, logit_chunk):
    x1 = i42
    x0 = x1
    x1 = ((x1 << jnp.uint32(_ROT[0])) | (x1 >> jnp.uint32(32 - _ROT[0]))) ^ x0
    for r in _ROT[1:4]:
        x0 = x0 + x1
        x1 = ((x1 << jnp.uint32(r)) | (x1 >> jnp.uint32(32 - r))) ^ x0
    for g in range(1, 5):
        x0 = x0 + jnp.uint32(_KS[g % 3])
        x1 = x1 + jnp.uint32((_KS[(g + 1) % 3] + g) & 0xFFFFFFFF)
        rr = _ROT[:4] if g % 2 == 0 else _ROT[4:]
        for r in rr:
            x0 = x0 + x1
            x1 = ((x1 << jnp.uint32(r)) | (x1 >> jnp.uint32(32 - r))) ^ x0
    x0 = x0 + jnp.uint32(_KS[2])
    x1 = x1 + jnp.uint32((_KS[0] + 5) & 0xFFFFFFFF)
    bits = x0 ^ x1

    fb = (bits >> jnp.uint32(9)) | jnp.uint32(0x3F800000)
    u = jax.lax.bitcast_convert_type(fb, jnp.float32) - jnp.float32(1.0)
    u = jnp.maximum(_TINY, u)
    return -jnp.log(-jnp.log(u)) + logit_chunk


def _gumbel_argmax_block(logits_ref, out_ref, *, vocab, rows):
    b = pl.program_id(0)
    n_full = vocab // _W
    tail = vocab - n_full * _W

    row = jax.lax.broadcasted_iota(jnp.uint32, (rows, _W), 0)
    col = jax.lax.broadcasted_iota(jnp.uint32, (rows, _W), 1)
    base = jnp.uint32(b) * jnp.uint32(rows) * jnp.uint32(vocab) + jnp.uint32(42)
    pre42 = row * jnp.uint32(vocab) + col + base
    col_i32 = col[0:1, :].astype(jnp.int32)

    def body(k, carry):
        best_s, best_i = carry
        off = k * _W
        score = _score_chunk(
            pre42 + jnp.uint32(off), logits_ref[:, pl.ds(off, _W)]
        )
        upd = score > best_s
        best_s = jnp.maximum(best_s, score)
        best_i = jnp.where(upd, col_i32 + off, best_i)
        return best_s, best_i

    init = (
        jnp.full((rows, _W), -jnp.inf, dtype=jnp.float32),
        jnp.zeros((rows, _W), dtype=jnp.int32),
    )
    best_s, best_i = jax.lax.fori_loop(0, n_full, body, init, unroll=False)

    m = jnp.max(best_s, axis=1, keepdims=True)
    cand = jnp.where(best_s == m, best_i, jnp.int32(0x7FFFFFFF))
    idx = jnp.min(cand, axis=1)
    mrow = m[:, 0]

    if tail:
        toff = n_full * _W
        trow = jax.lax.broadcasted_iota(jnp.uint32, (rows, tail), 0)
        tcol = jax.lax.broadcasted_iota(jnp.uint32, (rows, tail), 1)
        ti42 = trow * jnp.uint32(vocab) + tcol + base + jnp.uint32(toff)
        tscore = _score_chunk(ti42, logits_ref[:, pl.ds(toff, tail)])
        tm = jnp.max(tscore, axis=1, keepdims=True)
        tcand = jnp.where(
            tscore == tm, tcol.astype(jnp.int32) + toff, jnp.int32(0x7FFFFFFF)
        )
        tidx = jnp.min(tcand, axis=1)
        take_tail = tm[:, 0] > mrow
        idx = jnp.where(take_tail, tidx, idx)

    out_ref[0, 0, :] = idx


def kernel(logits):
    batch, vocab = logits.shape
    assert batch % _ROWS == 0
    grid = batch // _ROWS
    out = pl.pallas_call(
        functools.partial(_gumbel_argmax_block, vocab=vocab, rows=_ROWS),
        grid=(grid,),
        in_specs=[
            pl.BlockSpec((_ROWS, vocab), lambda b: (b, 0)),
        ],
        out_specs=pl.BlockSpec((1, 1, _ROWS), lambda b: (b, 0, 0)),
        out_shape=jax.ShapeDtypeStruct((grid, 1, _ROWS), jnp.int32),
        compiler_params=pltpu.CompilerParams(
            dimension_semantics=("arbitrary",),
        ),
    )(logits)
    return out.reshape(batch)


# speedup vs baseline: 1.5442x; 1.5246x over previous
import functools

import jax
import jax.numpy as jnp
import numpy as np
from jax.experimental import pallas as pl
from jax.experimental.pallas import tpu as pltpu

_ROWS = 8
_W = 512

_ROT = (13, 15, 26, 6, 17, 29, 16, 24)
_TINY = np.float32(np.finfo(np.float32).tiny)
_K1 = 0
_K2 = 42
_K3 = _K1 ^ _K2 ^ 0x1BD11BDA
_KS = (_K1, _K2, _K3)


def _score_chunk(i42, logit_chunk):
    x1 = i42
    x0 = x1
    x1 = ((x1 << jnp.uint32(_ROT[0])) | (x1 >> jnp.uint32(32 - _ROT[0]))) ^ x0
    for r in _ROT[1:4]:
        x0 = x0 + x1
        x1 = ((x1 << jnp.uint32(r)) | (x1 >> jnp.uint32(32 - r))) ^ x0
    for g in range(1, 5):
        x0 = x0 + jnp.uint32(_KS[g % 3])
        x1 = x1 + jnp.uint32((_KS[(g + 1) % 3] + g) & 0xFFFFFFFF)
        rr = _ROT[:4] if g % 2 == 0 else _ROT[4:]
        for r in rr:
            x0 = x0 + x1
            x1 = ((x1 << jnp.uint32(r)) | (x1 >> jnp.uint32(32 - r))) ^ x0
    x0 = x0 + jnp.uint32(_KS[2])
    x1 = x1 + jnp.uint32((_KS[0] + 5) & 0xFFFFFFFF)
    bits = x0 ^ x1

    fb = (bits >> jnp.uint32(9)) | jnp.uint32(0x3F800000)
    u = jax.lax.bitcast_convert_type(fb, jnp.float32) - jnp.float32(1.0)
    u = jnp.maximum(_TINY, u)
    return -jnp.log(-jnp.log(u)) + logit_chunk


def _gumbel_argmax_block(logits_ref, out_ref, *, vocab, rows):
    b = pl.program_id(0)
    n_full = vocab // _W
    tail = vocab - n_full * _W

    row = jax.lax.broadcasted_iota(jnp.uint32, (rows, _W), 0)
    col = jax.lax.broadcasted_iota(jnp.uint32, (rows, _W), 1)
    base = jnp.uint32(b) * jnp.uint32(rows) * jnp.uint32(vocab) + jnp.uint32(42)
    pre42 = row * jnp.uint32(vocab) + col + base
    col_i32 = col[0:1, :].astype(jnp.int32)

    def body(k, carry):
        best_s, best_i = carry
        off = k * _W
        score = _score_chunk(
            pre42 + jnp.uint32(off), logits_ref[:, pl.ds(off, _W)]
        )
        upd = score > best_s
        best_s = jnp.maximum(best_s, score)
        best_i = jnp.where(upd, col_i32 + off, best_i)
        return best_s, best_i

    init = (
        jnp.full((rows, _W), -jnp.inf, dtype=jnp.float32),
        jnp.zeros((rows, _W), dtype=jnp.int32),
    )
    best_s, best_i = jax.lax.fori_loop(0, n_full, body, init, unroll=4)

    m = jnp.max(best_s, axis=1, keepdims=True)
    cand = jnp.where(best_s == m, best_i, jnp.int32(0x7FFFFFFF))
    idx = jnp.min(cand, axis=1)
    mrow = m[:, 0]

    if tail:
        toff = n_full * _W
        trow = jax.lax.broadcasted_iota(jnp.uint32, (rows, tail), 0)
        tcol = jax.lax.broadcasted_iota(jnp.uint32, (rows, tail), 1)
        ti42 = trow * jnp.uint32(vocab) + tcol + base + jnp.uint32(toff)
        tscore = _score_chunk(ti42, logits_ref[:, pl.ds(toff, tail)])
        tm = jnp.max(tscore, axis=1, keepdims=True)
        tcand = jnp.where(
            tscore == tm, tcol.astype(jnp.int32) + toff, jnp.int32(0x7FFFFFFF)
        )
        tidx = jnp.min(tcand, axis=1)
        take_tail = tm[:, 0] > mrow
        idx = jnp.where(take_tail, tidx, idx)

    out_ref[0, 0, :] = idx


def kernel(logits):
    batch, vocab = logits.shape
    assert batch % _ROWS == 0
    grid = batch // _ROWS
    out = pl.pallas_call(
        functools.partial(_gumbel_argmax_block, vocab=vocab, rows=_ROWS),
        grid=(grid,),
        in_specs=[
            pl.BlockSpec((_ROWS, vocab), lambda b: (b, 0)),
        ],
        out_specs=pl.BlockSpec((1, 1, _ROWS), lambda b: (b, 0, 0)),
        out_shape=jax.ShapeDtypeStruct((grid, 1, _ROWS), jnp.int32),
        compiler_params=pltpu.CompilerParams(
            dimension_semantics=("arbitrary",),
        ),
    )(logits)
    return out.reshape(batch)


# speedup vs baseline: 1.5888x; 1.0289x over previous
import functools

import jax
import jax.numpy as jnp
import numpy as np
from jax.experimental import pallas as pl
from jax.experimental.pallas import tpu as pltpu

_ROWS = 8
_W = 512

_ROT = (13, 15, 26, 6, 17, 29, 16, 24)
_TINY = np.float32(np.finfo(np.float32).tiny)
_K1 = 0
_K2 = 42
_K3 = _K1 ^ _K2 ^ 0x1BD11BDA
_KS = (_K1, _K2, _K3)


def _score_chunk(i42, logit_chunk):
    x1 = i42
    x0 = x1
    x1 = ((x1 << jnp.uint32(_ROT[0])) | (x1 >> jnp.uint32(32 - _ROT[0]))) ^ x0
    for r in _ROT[1:4]:
        x0 = x0 + x1
        x1 = ((x1 << jnp.uint32(r)) | (x1 >> jnp.uint32(32 - r))) ^ x0
    for g in range(1, 5):
        x0 = x0 + jnp.uint32(_KS[g % 3])
        x1 = x1 + jnp.uint32((_KS[(g + 1) % 3] + g) & 0xFFFFFFFF)
        rr = _ROT[:4] if g % 2 == 0 else _ROT[4:]
        for r in rr:
            x0 = x0 + x1
            x1 = ((x1 << jnp.uint32(r)) | (x1 >> jnp.uint32(32 - r))) ^ x0
    x0 = x0 + jnp.uint32(_KS[2])
    x1 = x1 + jnp.uint32((_KS[0] + 5) & 0xFFFFFFFF)
    bits = x0 ^ x1

    fb = (bits >> jnp.uint32(9)) | jnp.uint32(0x3F800000)
    u = jax.lax.bitcast_convert_type(fb, jnp.float32) - jnp.float32(1.0)
    u = jnp.maximum(_TINY, u)
    return -jnp.log(-jnp.log(u)) + logit_chunk


def _gumbel_argmax_block(logits_ref, out_ref, *, vocab, rows):
    b = pl.program_id(0)
    n_full = vocab // _W
    tail = vocab - n_full * _W

    row = jax.lax.broadcasted_iota(jnp.uint32, (rows, _W), 0)
    col = jax.lax.broadcasted_iota(jnp.uint32, (rows, _W), 1)
    base = jnp.uint32(b) * jnp.uint32(rows) * jnp.uint32(vocab) + jnp.uint32(42)
    pre42 = row * jnp.uint32(vocab) + col + base
    col_i32 = col[0:1, :].astype(jnp.int32)

    def body(k, carry):
        best_s, best_i = carry
        off = k * _W
        score = _score_chunk(
            pre42 + jnp.uint32(off), logits_ref[:, pl.ds(off, _W)]
        )
        upd = score > best_s
        best_s = jnp.maximum(best_s, score)
        best_i = jnp.where(upd, col_i32 + off, best_i)
        return best_s, best_i

    init = (
        jnp.full((rows, _W), -jnp.inf, dtype=jnp.float32),
        jnp.zeros((rows, _W), dtype=jnp.int32),
    )
    best_s, best_i = jax.lax.fori_loop(0, n_full, body, init, unroll=8)

    m = jnp.max(best_s, axis=1, keepdims=True)
    cand = jnp.where(best_s == m, best_i, jnp.int32(0x7FFFFFFF))
    idx = jnp.min(cand, axis=1)
    mrow = m[:, 0]

    if tail:
        toff = n_full * _W
        trow = jax.lax.broadcasted_iota(jnp.uint32, (rows, tail), 0)
        tcol = jax.lax.broadcasted_iota(jnp.uint32, (rows, tail), 1)
        ti42 = trow * jnp.uint32(vocab) + tcol + base + jnp.uint32(toff)
        tscore = _score_chunk(ti42, logits_ref[:, pl.ds(toff, tail)])
        tm = jnp.max(tscore, axis=1, keepdims=True)
        tcand = jnp.where(
            tscore == tm, tcol.astype(jnp.int32) + toff, jnp.int32(0x7FFFFFFF)
        )
        tidx = jnp.min(tcand, axis=1)
        take_tail = tm[:, 0] > mrow
        idx = jnp.where(take_tail, tidx, idx)

    out_ref[0, 0, :] = idx


def kernel(logits):
    batch, vocab = logits.shape
    assert batch % _ROWS == 0
    grid = batch // _ROWS
    out = pl.pallas_call(
        functools.partial(_gumbel_argmax_block, vocab=vocab, rows=_ROWS),
        grid=(grid,),
        in_specs=[
            pl.BlockSpec((_ROWS, vocab), lambda b: (b, 0)),
        ],
        out_specs=pl.BlockSpec((1, 1, _ROWS), lambda b: (b, 0, 0)),
        out_shape=jax.ShapeDtypeStruct((grid, 1, _ROWS), jnp.int32),
        compiler_params=pltpu.CompilerParams(
            dimension_semantics=("arbitrary",),
        ),
    )(logits)
    return out.reshape(batch)


# speedup vs baseline: 1.6097x; 1.0131x over previous
import functools

import jax
import jax.numpy as jnp
import numpy as np
from jax.experimental import pallas as pl
from jax.experimental.pallas import tpu as pltpu

_ROWS = 8
_W = 1024

_ROT = (13, 15, 26, 6, 17, 29, 16, 24)
_TINY = np.float32(np.finfo(np.float32).tiny)
_K1 = 0
_K2 = 42
_K3 = _K1 ^ _K2 ^ 0x1BD11BDA
_KS = (_K1, _K2, _K3)


def _score_chunk(i42, logit_chunk):
    x1 = i42
    x0 = x1
    x1 = ((x1 << jnp.uint32(_ROT[0])) | (x1 >> jnp.uint32(32 - _ROT[0]))) ^ x0
    for r in _ROT[1:4]:
        x0 = x0 + x1
        x1 = ((x1 << jnp.uint32(r)) | (x1 >> jnp.uint32(32 - r))) ^ x0
    for g in range(1, 5):
        x0 = x0 + jnp.uint32(_KS[g % 3])
        x1 = x1 + jnp.uint32((_KS[(g + 1) % 3] + g) & 0xFFFFFFFF)
        rr = _ROT[:4] if g % 2 == 0 else _ROT[4:]
        for r in rr:
            x0 = x0 + x1
            x1 = ((x1 << jnp.uint32(r)) | (x1 >> jnp.uint32(32 - r))) ^ x0
    x0 = x0 + jnp.uint32(_KS[2])
    x1 = x1 + jnp.uint32((_KS[0] + 5) & 0xFFFFFFFF)
    bits = x0 ^ x1

    fb = (bits >> jnp.uint32(9)) | jnp.uint32(0x3F800000)
    u = jax.lax.bitcast_convert_type(fb, jnp.float32) - jnp.float32(1.0)
    u = jnp.maximum(_TINY, u)
    return -jnp.log(-jnp.log(u)) + logit_chunk


def _gumbel_argmax_block(logits_ref, out_ref, *, vocab, rows):
    b = pl.program_id(0)
    n_full = vocab // _W
    tail = vocab - n_full * _W

    row = jax.lax.broadcasted_iota(jnp.uint32, (rows, _W), 0)
    col = jax.lax.broadcasted_iota(jnp.uint32, (rows, _W), 1)
    base = jnp.uint32(b) * jnp.uint32(rows) * jnp.uint32(vocab) + jnp.uint32(42)
    pre42 = row * jnp.uint32(vocab) + col + base
    col_i32 = col[0:1, :].astype(jnp.int32)

    def body(k, carry):
        best_s, best_i = carry
        off = k * _W
        score = _score_chunk(
            pre42 + jnp.uint32(off), logits_ref[:, pl.ds(off, _W)]
        )
        upd = score > best_s
        best_s = jnp.maximum(best_s, score)
        best_i = jnp.where(upd, col_i32 + off, best_i)
        return best_s, best_i

    init = (
        jnp.full((rows, _W), -jnp.inf, dtype=jnp.float32),
        jnp.zeros((rows, _W), dtype=jnp.int32),
    )
    best_s, best_i = jax.lax.fori_loop(0, n_full, body, init, unroll=4)

    m = jnp.max(best_s, axis=1, keepdims=True)
    cand = jnp.where(best_s == m, best_i, jnp.int32(0x7FFFFFFF))
    idx = jnp.min(cand, axis=1)
    mrow = m[:, 0]

    if tail:
        toff = n_full * _W
        trow = jax.lax.broadcasted_iota(jnp.uint32, (rows, tail), 0)
        tcol = jax.lax.broadcasted_iota(jnp.uint32, (rows, tail), 1)
        ti42 = trow * jnp.uint32(vocab) + tcol + base + jnp.uint32(toff)
        tscore = _score_chunk(ti42, logits_ref[:, pl.ds(toff, tail)])
        tm = jnp.max(tscore, axis=1, keepdims=True)
        tcand = jnp.where(
            tscore == tm, tcol.astype(jnp.int32) + toff, jnp.int32(0x7FFFFFFF)
        )
        tidx = jnp.min(tcand, axis=1)
        take_tail = tm[:, 0] > mrow
        idx = jnp.where(take_tail, tidx, idx)

    out_ref[0, 0, :] = idx


def kernel(logits):
    batch, vocab = logits.shape
    assert batch % _ROWS == 0
    grid = batch // _ROWS
    out = pl.pallas_call(
        functools.partial(_gumbel_argmax_block, vocab=vocab, rows=_ROWS),
        grid=(grid,),
        in_specs=[
            pl.BlockSpec((_ROWS, vocab), lambda b: (b, 0)),
        ],
        out_specs=pl.BlockSpec((1, 1, _ROWS), lambda b: (b, 0, 0)),
        out_shape=jax.ShapeDtypeStruct((grid, 1, _ROWS), jnp.int32),
        compiler_params=pltpu.CompilerParams(
            dimension_semantics=("arbitrary",),
        ),
    )(logits)
    return out.reshape(batch)


# speedup vs baseline: 1.6162x; 1.0041x over previous
import functools

import jax
import jax.numpy as jnp
import numpy as np
from jax.experimental import pallas as pl
from jax.experimental.pallas import tpu as pltpu

_ROWS = 8
_W = 1024

_ROT = (13, 15, 26, 6, 17, 29, 16, 24)
_TINY = np.float32(np.finfo(np.float32).tiny)
_K1 = 0
_K2 = 42
_K3 = _K1 ^ _K2 ^ 0x1BD11BDA
_KS = (_K1, _K2, _K3)


def _score_chunk(i42, logit_chunk):
    x1 = i42
    x0 = x1
    x1 = ((x1 << jnp.uint32(_ROT[0])) | (x1 >> jnp.uint32(32 - _ROT[0]))) ^ x0
    for r in _ROT[1:4]:
        x0 = x0 + x1
        x1 = ((x1 << jnp.uint32(r)) | (x1 >> jnp.uint32(32 - r))) ^ x0
    for g in range(1, 5):
        x0 = x0 + jnp.uint32(_KS[g % 3])
        x1 = x1 + jnp.uint32((_KS[(g + 1) % 3] + g) & 0xFFFFFFFF)
        rr = _ROT[:4] if g % 2 == 0 else _ROT[4:]
        for r in rr:
            x0 = x0 + x1
            x1 = ((x1 << jnp.uint32(r)) | (x1 >> jnp.uint32(32 - r))) ^ x0
    x0 = x0 + jnp.uint32(_KS[2])
    x1 = x1 + jnp.uint32((_KS[0] + 5) & 0xFFFFFFFF)
    bits = x0 ^ x1

    fb = (bits >> jnp.uint32(9)) | jnp.uint32(0x3F800000)
    u = jax.lax.bitcast_convert_type(fb, jnp.float32) - jnp.float32(1.0)
    u = jnp.maximum(_TINY, u)
    return -jnp.log(-jnp.log(u)) + logit_chunk


def _gumbel_argmax_block(logits_ref, out_ref, *, vocab, rows):
    b = pl.program_id(0)
    n_full = vocab // _W
    tail = vocab - n_full * _W

    row = jax.lax.broadcasted_iota(jnp.uint32, (rows, _W), 0)
    col = jax.lax.broadcasted_iota(jnp.uint32, (rows, _W), 1)
    base = jnp.uint32(b) * jnp.uint32(rows) * jnp.uint32(vocab) + jnp.uint32(42)
    pre42 = row * jnp.uint32(vocab) + col + base
    col_i32 = col[0:1, :].astype(jnp.int32)

    def body(k, carry):
        best_s, best_i = carry
        off = k * _W
        score = _score_chunk(
            pre42 + jnp.uint32(off), logits_ref[:, pl.ds(off, _W)]
        )
        upd = score > best_s
        best_s = jnp.maximum(best_s, score)
        best_i = jnp.where(upd, col_i32 + off, best_i)
        return best_s, best_i

    init = (
        jnp.full((rows, _W), -jnp.inf, dtype=jnp.float32),
        jnp.zeros((rows, _W), dtype=jnp.int32),
    )
    best_s, best_i = jax.lax.fori_loop(0, n_full, body, init, unroll=6)

    m = jnp.max(best_s, axis=1, keepdims=True)
    cand = jnp.where(best_s == m, best_i, jnp.int32(0x7FFFFFFF))
    idx = jnp.min(cand, axis=1)
    mrow = m[:, 0]

    if tail:
        toff = n_full * _W
        trow = jax.lax.broadcasted_iota(jnp.uint32, (rows, tail), 0)
        tcol = jax.lax.broadcasted_iota(jnp.uint32, (rows, tail), 1)
        ti42 = trow * jnp.uint32(vocab) + tcol + base + jnp.uint32(toff)
        tscore = _score_chunk(ti42, logits_ref[:, pl.ds(toff, tail)])
        tm = jnp.max(tscore, axis=1, keepdims=True)
        tcand = jnp.where(
            tscore == tm, tcol.astype(jnp.int32) + toff, jnp.int32(0x7FFFFFFF)
        )
        tidx = jnp.min(tcand, axis=1)
        take_tail = tm[:, 0] > mrow
        idx = jnp.where(take_tail, tidx, idx)

    out_ref[0, 0, :] = idx


def kernel(logits):
    batch, vocab = logits.shape
    assert batch % _ROWS == 0
    grid = batch // _ROWS
    out = pl.pallas_call(
        functools.partial(_gumbel_argmax_block, vocab=vocab, rows=_ROWS),
        grid=(grid,),
        in_specs=[
            pl.BlockSpec((_ROWS, vocab), lambda b: (b, 0)),
        ],
        out_specs=pl.BlockSpec((1, 1, _ROWS), lambda b: (b, 0, 0)),
        out_shape=jax.ShapeDtypeStruct((grid, 1, _ROWS), jnp.int32),
        compiler_params=pltpu.CompilerParams(
            dimension_semantics=("arbitrary",),
        ),
    )(logits)
    return out.reshape(batch)
